# Initial kernel scaffold; baseline (speedup 1.0000x reference)
import jax
import jax.numpy as jnp
from jax.experimental import pallas as pl


def kernel(edge_index, edge_type, W1, root1, bias1, W2, att_q, att_k, bias2):
    raise NotImplementedError("write your pallas kernel here")


# speedup vs baseline: 26.7486x; 26.7486x over previous
import functools

import jax
import jax.numpy as jnp
from jax import lax
from jax.experimental import pallas as pl
from jax.experimental.pallas import tpu as pltpu
from jax.experimental.pallas import tpu_sc as plsc

NC = 2
NS = 16
NW = NC * NS
L = 16
C = 128

_mesh = functools.partial(
    plsc.VectorSubcoreMesh,
    core_axis_name="c", subcore_axis_name="s",
    num_cores=NC, num_subcores=NS,
)


def _wid_and_span(n_chunks):
    base, extra = divmod(n_chunks, NW)

    def get(wid):
        start = wid * base + jnp.minimum(wid, extra)
        nj = base + (wid < extra).astype(jnp.int32)
        return start, nj

    return base, extra, get


def _zero_fill(ref, nrows, ncols):
    z = jnp.zeros((L,), jnp.float32)

    def body(r, _):
        for t in range(ncols // L):
            ref[r, pl.ds(t * L, L)] = z
        return 0

    lax.fori_loop(0, nrows, body, 0)


def _row_partition(N):
    rpt = ((N + NS - 1) // NS + 7) // 8 * 8
    last = N - (NS - 1) * rpt
    assert 0 < last <= rpt and last % 8 == 0
    full = last // C
    return rpt, full, rpt - full * C, last - full * C


def _foreach_slab(sid, N, cb):
    rpt, full, tail_main, tail_last = _row_partition(N)
    r0 = sid * rpt
    for i in range(full):
        cb(r0 + i * C, C)
    if tail_main:
        @pl.when(sid < NS - 1)
        def _():
            cb(r0 + full * C, tail_main)
    if tail_last:
        @pl.when(sid == NS - 1)
        def _():
            cb(r0 + full * C, tail_last)


def _pack8_rows(N):
    per = NS * 8
    return ((N + 8 - 1) // 8 + per - 1) // per * per


@functools.cache
def _make_k1(N, H, E, R):
    n_chunks = E // C
    base, extra, get_span = _wid_and_span(n_chunks)
    n8 = _pack8_rows(N)

    def body(src_h, rel_h, dst_h, w1_h, agg_out, deg_out,
             sidx_v, ridx_v, didx_v, gidx_v, didx8_v, rows_v, onerow_v,
             acc_s, deg8_s, sem, sem_idx):
        cid = lax.axis_index("c")
        sid = lax.axis_index("s")
        wid = sid * NC + cid
        start, nj = get_span(wid)
        iota = lax.iota(jnp.int32, L)
        onehot_f = jnp.where(iota == 0, jnp.float32(1.0), jnp.float32(0.0))

        _zero_fill(rows_v, C, H)
        _zero_fill(onerow_v, C, H)

        def zero_slab(r, n):
            pltpu.async_copy(rows_v.at[pl.ds(0, n)], acc_s.at[pl.ds(r, n)],
                             sem).wait()

        _foreach_slab(sid, N, zero_slab)
        pltpu.async_copy(onerow_v.at[pl.ds(0, n8 // NS)],
                         deg8_s.at[pl.ds(sid * (n8 // NS), n8 // NS)],
                         sem).wait()

        plsc.subcore_barrier()

        def chunk(j, _):
            e0 = (start + j) * C
            cps = pltpu.async_copy(src_h.at[pl.ds(e0, C)], sidx_v, sem_idx)
            cpr = pltpu.async_copy(rel_h.at[pl.ds(e0, C)], ridx_v, sem_idx)
            cpd = pltpu.async_copy(dst_h.at[pl.ds(e0, C)], didx_v, sem_idx)
            cps.wait()
            cpr.wait()
            cpd.wait()
            for t in range(C // L):
                sl = pl.ds(t * L, L)
                gidx_v[sl] = ridx_v[sl] * N + sidx_v[sl]
                didx8_v[sl] = lax.shift_right_logical(didx_v[sl], 3)
            cpw = pltpu.async_copy(w1_h.at[gidx_v], rows_v, sem)

            def build(g, _):
                dm8v = lax.bitwise_and(didx_v[pl.ds(g * L, L)], 7)
                for p in range(L):
                    dm8 = dm8v[p]
                    e = g * L + p
                    for s in range(H // L):
                        sel = jnp.where(dm8 == s, jnp.float32(1.0),
                                        jnp.float32(0.0))
                        onerow_v[e, pl.ds(s * L, L)] = onehot_f * sel
                return 0

            lax.fori_loop(0, C // L, build, 0)
            cpw.wait()
            pltpu.sync_copy(rows_v, acc_s.at[didx_v], add=True)
            pltpu.sync_copy(onerow_v, deg8_s.at[didx8_v], add=True)
            return 0

        lax.fori_loop(0, nj, chunk, 0)

        plsc.subcore_barrier()

        def out_slab(r, n):
            pltpu.async_copy(acc_s.at[pl.ds(r, n)], rows_v.at[pl.ds(0, n)],
                             sem).wait()
            pltpu.sync_copy(rows_v.at[pl.ds(0, n)],
                            agg_out.at[pl.ds(cid * N + r, n)])

        _foreach_slab(sid, N, out_slab)
        r8 = sid * (n8 // NS)
        pltpu.async_copy(deg8_s.at[pl.ds(r8, n8 // NS)],
                         onerow_v.at[pl.ds(0, n8 // NS)], sem).wait()
        pltpu.sync_copy(onerow_v.at[pl.ds(0, n8 // NS)],
                        deg_out.at[pl.ds(cid * n8 + r8, n8 // NS)])

    return pl.kernel(
        body,
        out_type=[
            jax.ShapeDtypeStruct((NC * N, H), jnp.float32),
            jax.ShapeDtypeStruct((NC * n8, H), jnp.float32),
        ],
        mesh=_mesh(),
        scratch_types=[
            pltpu.VMEM((C,), jnp.int32),
            pltpu.VMEM((C,), jnp.int32),
            pltpu.VMEM((C,), jnp.int32),
            pltpu.VMEM((C,), jnp.int32),
            pltpu.VMEM((C,), jnp.int32),
            pltpu.VMEM((C, H), jnp.float32),
            pltpu.VMEM((C, H), jnp.float32),
            pltpu.VMEM_SHARED((N, H), jnp.float32),
            pltpu.VMEM_SHARED((n8, H), jnp.float32),
            pltpu.SemaphoreType.DMA,
            pltpu.SemaphoreType.DMA,
        ],
    )


@functools.cache
def _make_k2(N, H, R, BN):
    nb = N // BN

    def body(agg_ref, deg_ref, root_ref, b1_ref, w2_ref, aq_ref, ak_ref,
             xw_ref, qk_ref):
        agg = agg_ref[0] + agg_ref[1]
        deg = jnp.sum(deg_ref[0] + deg_ref[1], axis=-1, keepdims=True)
        x = agg / jnp.maximum(deg, 1.0) + root_ref[...] + b1_ref[...]
        x = jnp.where(x > 0, x, 0.01 * x)
        w2 = w2_ref[...]
        mq = lax.dot_general(w2, aq_ref[0], (((2,), (0,)), ((), ())),
                             preferred_element_type=jnp.float32)
        mk = lax.dot_general(w2, ak_ref[0], (((2,), (0,)), ((), ())),
                             preferred_element_type=jnp.float32)
        m = jnp.concatenate([mq, mk], axis=0)
        qk_ref[...] = lax.dot_general(x, m, (((1,), (1,)), ((), ())),
                                      preferred_element_type=jnp.float32)
        for r in range(R):
            xw_ref[r] = jnp.dot(x, w2[r], preferred_element_type=jnp.float32)

    return pl.pallas_call(
        body,
        grid=(nb,),
        in_specs=[
            pl.BlockSpec((NC, BN, H), lambda i: (0, i, 0)),
            pl.BlockSpec((NC, BN, L), lambda i: (0, i, 0)),
            pl.BlockSpec((BN, H), lambda i: (i, 0)),
            pl.BlockSpec((1, H), lambda i: (0, 0)),
            pl.BlockSpec((R, H, H), lambda i: (0, 0, 0)),
            pl.BlockSpec((1, H), lambda i: (0, 0)),
            pl.BlockSpec((1, H), lambda i: (0, 0)),
        ],
        out_specs=[
            pl.BlockSpec((R, BN, H), lambda i: (0, i, 0)),
            pl.BlockSpec((BN, 2 * R), lambda i: (i, 0)),
        ],
        out_shape=[
            jax.ShapeDtypeStruct((R, N, H), jnp.float32),
            jax.ShapeDtypeStruct((N, 2 * R), jnp.float32),
        ],
    )


@functools.cache
def _make_k3(N, H, E, R):
    n_chunks = E // C
    base, extra, get_span = _wid_and_span(n_chunks)
    n8 = _pack8_rows(N)

    def body(src_h, rel_h, dst_h, xw_h, qk_h, num_out, den_out,
             sidx_v, ridx_v, didx_v, gidx_v, qidx_v, kidx_v, didx8_v,
             qe_t, ke_t, rows_v, exrow_v, ex_v,
             acc_s, den8_s, sem1, sem2, sem3, sem_idx):
        cid = lax.axis_index("c")
        sid = lax.axis_index("s")
        wid = sid * NC + cid
        start, nj = get_span(wid)
        iota = lax.iota(jnp.int32, L)
        onehot_f = jnp.where(iota == 0, jnp.float32(1.0), jnp.float32(0.0))

        _zero_fill(rows_v, C, H)
        _zero_fill(exrow_v, C, H)

        def zero_slab(r, n):
            pltpu.async_copy(rows_v.at[pl.ds(0, n)], acc_s.at[pl.ds(r, n)],
                             sem1).wait()

        _foreach_slab(sid, N, zero_slab)
        pltpu.async_copy(exrow_v.at[pl.ds(0, n8 // NS)],
                         den8_s.at[pl.ds(sid * (n8 // NS), n8 // NS)],
                         sem1).wait()

        plsc.subcore_barrier()

        def chunk(j, _):
            e0 = (start + j) * C
            cps = pltpu.async_copy(src_h.at[pl.ds(e0, C)], sidx_v, sem_idx)
            cpr = pltpu.async_copy(rel_h.at[pl.ds(e0, C)], ridx_v, sem_idx)
            cpd = pltpu.async_copy(dst_h.at[pl.ds(e0, C)], didx_v, sem_idx)
            cps.wait()
            cpr.wait()
            cpd.wait()
            for t in range(C // L):
                sl = pl.ds(t * L, L)
                r = ridx_v[sl]
                gidx_v[sl] = r * N + sidx_v[sl]
                qidx_v[sl] = sidx_v[sl] * (2 * R) + r
                kidx_v[sl] = didx_v[sl] * (2 * R) + (r + R)
                didx8_v[sl] = lax.shift_right_logical(didx_v[sl], 3)
            cp1 = pltpu.async_copy(qk_h.at[qidx_v], qe_t, sem1)
            cp2 = pltpu.async_copy(qk_h.at[kidx_v], ke_t, sem2)
            cp3 = pltpu.async_copy(xw_h.at[gidx_v], rows_v, sem3)
            cp1.wait()
            cp2.wait()
            for t in range(C // L):
                sl = pl.ds(t * L, L)
                lg = qe_t[sl] + ke_t[sl]
                lg = jnp.maximum(lg, 0.2 * lg)
                ex_v[sl] = jnp.exp(lg)
            cp3.wait()

            def scale(g, _):
                exv = ex_v[pl.ds(g * L, L)]
                dm8v = lax.bitwise_and(didx_v[pl.ds(g * L, L)], 7)
                for p in range(L):
                    sca = exv[p]
                    dm8 = dm8v[p]
                    e = g * L + p
                    for t in range(H // L):
                        sl2 = pl.ds(t * L, L)
                        rows_v[e, sl2] = rows_v[e, sl2] * sca
                        sel = jnp.where(dm8 == t, sca, jnp.float32(0.0))
                        exrow_v[e, sl2] = onehot_f * sel
                return 0

            lax.fori_loop(0, C // L, scale, 0)
            pltpu.sync_copy(rows_v, acc_s.at[didx_v], add=True)
            pltpu.sync_copy(exrow_v, den8_s.at[didx8_v], add=True)
            return 0

        lax.fori_loop(0, nj, chunk, 0)

        plsc.subcore_barrier()

        def out_slab(r, n):
            pltpu.async_copy(acc_s.at[pl.ds(r, n)], rows_v.at[pl.ds(0, n)],
                             sem1).wait()
            pltpu.sync_copy(rows_v.at[pl.ds(0, n)],
                            num_out.at[pl.ds(cid * N + r, n)])

        _foreach_slab(sid, N, out_slab)
        r8 = sid * (n8 // NS)
        pltpu.async_copy(den8_s.at[pl.ds(r8, n8 // NS)],
                         exrow_v.at[pl.ds(0, n8 // NS)], sem1).wait()
        pltpu.sync_copy(exrow_v.at[pl.ds(0, n8 // NS)],
                        den_out.at[pl.ds(cid * n8 + r8, n8 // NS)])

    return pl.kernel(
        body,
        out_type=[
            jax.ShapeDtypeStruct((NC * N, H), jnp.float32),
            jax.ShapeDtypeStruct((NC * n8, H), jnp.float32),
        ],
        mesh=_mesh(),
        scratch_types=[
            pltpu.VMEM((C,), jnp.int32),
            pltpu.VMEM((C,), jnp.int32),
            pltpu.VMEM((C,), jnp.int32),
            pltpu.VMEM((C,), jnp.int32),
            pltpu.VMEM((C,), jnp.int32),
            pltpu.VMEM((C,), jnp.int32),
            pltpu.VMEM((C,), jnp.int32),
            pltpu.VMEM((C,), jnp.float32),
            pltpu.VMEM((C,), jnp.float32),
            pltpu.VMEM((C, H), jnp.float32),
            pltpu.VMEM((C, H), jnp.float32),
            pltpu.VMEM((C,), jnp.float32),
            pltpu.VMEM_SHARED((N, H), jnp.float32),
            pltpu.VMEM_SHARED((n8, H), jnp.float32),
            pltpu.SemaphoreType.DMA,
            pltpu.SemaphoreType.DMA,
            pltpu.SemaphoreType.DMA,
            pltpu.SemaphoreType.DMA,
        ],
    )


@functools.cache
def _make_k4(N, H, BN):
    nb = N // BN

    def body(num_ref, den_ref, b2_ref, out_ref):
        num = num_ref[0] + num_ref[1]
        den = jnp.sum(den_ref[0] + den_ref[1], axis=-1, keepdims=True)
        out_ref[...] = num / jnp.maximum(den, 1e-16) + b2_ref[...]

    return pl.pallas_call(
        body,
        grid=(nb,),
        in_specs=[
            pl.BlockSpec((NC, BN, H), lambda i: (0, i, 0)),
            pl.BlockSpec((NC, BN, L), lambda i: (0, i, 0)),
            pl.BlockSpec((1, H), lambda i: (0, 0)),
        ],
        out_specs=pl.BlockSpec((BN, H), lambda i: (i, 0)),
        out_shape=jax.ShapeDtypeStruct((N, H), jnp.float32),
    )


def kernel(edge_index, edge_type, W1, root1, bias1, W2, att_q, att_k, bias2):
    R, Nw1, H = W1.shape
    N = root1.shape[0]
    E = edge_type.shape[0]
    assert Nw1 == N and E % C == 0 and N % NS == 0 and H % L == 0 and 2 * R == L

    src = edge_index[0]
    dst = edge_index[1]
    w1_flat = W1.reshape(R * N, H)

    n8 = _pack8_rows(N)
    k1 = _make_k1(N, H, E, R)
    aggP, degP = k1(src, edge_type, dst, w1_flat)

    BN = 1000
    k2 = _make_k2(N, H, R, BN)
    xw, qk = k2(aggP.reshape(NC, N, H), degP.reshape(NC, n8 * 8, L),
                root1, bias1.reshape(1, H), W2,
                att_q.reshape(1, H), att_k.reshape(1, H))

    k3 = _make_k3(N, H, E, R)
    numP, denP = k3(src, edge_type, dst, xw.reshape(R * N, H),
                    qk.reshape(N * 2 * R))

    k4 = _make_k4(N, H, BN)
    out = k4(numP.reshape(NC, N, H), denP.reshape(NC, n8 * 8, L),
             bias2.reshape(1, H))
    return out


# speedup vs baseline: 29.5728x; 1.1056x over previous
import functools

import jax
import jax.numpy as jnp
from jax import lax
from jax.experimental import pallas as pl
from jax.experimental.pallas import tpu as pltpu
from jax.experimental.pallas import tpu_sc as plsc

NC = 2
NS = 16
NW = NC * NS
L = 16
C = 128

_mesh = functools.partial(
    plsc.VectorSubcoreMesh,
    core_axis_name="c", subcore_axis_name="s",
    num_cores=NC, num_subcores=NS,
)


def _wid_and_span(n_chunks):
    base, extra = divmod(n_chunks, NW)

    def get(wid):
        start = wid * base + jnp.minimum(wid, extra)
        nj = base + (wid < extra).astype(jnp.int32)
        return start, nj

    return base, extra, get


def _zero_fill(ref, nrows, ncols):
    z = jnp.zeros((L,), jnp.float32)

    def body(r, _):
        for t in range(ncols // L):
            ref[r, pl.ds(t * L, L)] = z
        return 0

    lax.fori_loop(0, nrows, body, 0)


def _row_partition(N):
    rpt = ((N + NS - 1) // NS + 7) // 8 * 8
    last = N - (NS - 1) * rpt
    assert 0 < last <= rpt and last % 8 == 0
    full = last // C
    return rpt, full, rpt - full * C, last - full * C


def _foreach_slab(sid, N, cb):
    rpt, full, tail_main, tail_last = _row_partition(N)
    r0 = sid * rpt
    for i in range(full):
        cb(r0 + i * C, C)
    if tail_main:
        @pl.when(sid < NS - 1)
        def _():
            cb(r0 + full * C, tail_main)
    if tail_last:
        @pl.when(sid == NS - 1)
        def _():
            cb(r0 + full * C, tail_last)


def _pack8_rows(N):
    per = NS * 8
    return ((N + 8 - 1) // 8 + per - 1) // per * per


@functools.cache
def _make_k1(N, H, E, R):
    n_chunks = E // C
    base, extra, get_span = _wid_and_span(n_chunks)
    n8 = _pack8_rows(N)

    def body(src_h, rel_h, dst_h, w1_h, agg_out, deg_out,
             sidx_v, ridx_v, didx_v, gidx_v, didx8_v, didxs_v,
             rows_v, onerow_v, acc_s, deg8_s, sem, sem2, sem_idx):
        cid = lax.axis_index("c")
        sid = lax.axis_index("s")
        wid = sid * NC + cid
        start, nj = get_span(wid)
        iota = lax.iota(jnp.int32, L)
        onehot_f = jnp.where(iota == 0, jnp.float32(1.0), jnp.float32(0.0))

        _zero_fill(rows_v, C, H)
        _zero_fill(onerow_v, C, H)

        def zero_slab(r, n):
            pltpu.async_copy(rows_v.at[pl.ds(0, n)], acc_s.at[pl.ds(r, n)],
                             sem).wait()

        _foreach_slab(sid, N, zero_slab)
        pltpu.async_copy(onerow_v.at[pl.ds(0, n8 // NS)],
                         deg8_s.at[pl.ds(sid * (n8 // NS), n8 // NS)],
                         sem).wait()

        plsc.subcore_barrier()

        def fire_idx(j):
            e0 = (start + j) * C
            pltpu.async_copy(src_h.at[pl.ds(e0, C)], sidx_v, sem_idx)
            pltpu.async_copy(rel_h.at[pl.ds(e0, C)], ridx_v, sem_idx)
            pltpu.async_copy(dst_h.at[pl.ds(e0, C)], didx_v, sem_idx)

        def drain_idx():
            pltpu.make_async_copy(src_h.at[pl.ds(0, C)], sidx_v, sem_idx).wait()
            pltpu.make_async_copy(rel_h.at[pl.ds(0, C)], ridx_v, sem_idx).wait()
            pltpu.make_async_copy(dst_h.at[pl.ds(0, C)], didx_v, sem_idx).wait()

        fire_idx(0)

        def chunk(j, _):
            drain_idx()
            for t in range(C // L):
                sl = pl.ds(t * L, L)
                gidx_v[sl] = ridx_v[sl] * N + sidx_v[sl]
                didx8_v[sl] = lax.shift_right_logical(didx_v[sl], 3)
                didxs_v[sl] = didx_v[sl]
            fire_idx(jnp.minimum(j + 1, nj - 1))
            cpw = pltpu.async_copy(w1_h.at[gidx_v], rows_v, sem)

            def build(g, _):
                dm8v = lax.bitwise_and(didxs_v[pl.ds(g * L, L)], 7)
                for p in range(L):
                    dm8 = dm8v[p]
                    e = g * L + p
                    for s in range(H // L):
                        sel = jnp.where(dm8 == s, jnp.float32(1.0),
                                        jnp.float32(0.0))
                        onerow_v[e, pl.ds(s * L, L)] = onehot_f * sel
                return 0

            lax.fori_loop(0, C // L, build, 0)
            cpw.wait()
            ca = pltpu.async_copy(rows_v, acc_s.at[didxs_v], sem, add=True)
            cb = pltpu.async_copy(onerow_v, deg8_s.at[didx8_v], sem2,
                                  add=True)
            ca.wait()
            cb.wait()
            return 0

        lax.fori_loop(0, nj, chunk, 0)
        drain_idx()

        plsc.subcore_barrier()

        def out_slab(r, n):
            pltpu.async_copy(acc_s.at[pl.ds(r, n)], rows_v.at[pl.ds(0, n)],
                             sem).wait()
            pltpu.sync_copy(rows_v.at[pl.ds(0, n)],
                            agg_out.at[pl.ds(cid * N + r, n)])

        _foreach_slab(sid, N, out_slab)
        r8 = sid * (n8 // NS)
        pltpu.async_copy(deg8_s.at[pl.ds(r8, n8 // NS)],
                         onerow_v.at[pl.ds(0, n8 // NS)], sem).wait()
        pltpu.sync_copy(onerow_v.at[pl.ds(0, n8 // NS)],
                        deg_out.at[pl.ds(cid * n8 + r8, n8 // NS)])

    return pl.kernel(
        body,
        out_type=[
            jax.ShapeDtypeStruct((NC * N, H), jnp.float32),
            jax.ShapeDtypeStruct((NC * n8, H), jnp.float32),
        ],
        mesh=_mesh(),
        scratch_types=[
            pltpu.VMEM((C,), jnp.int32),
            pltpu.VMEM((C,), jnp.int32),
            pltpu.VMEM((C,), jnp.int32),
            pltpu.VMEM((C,), jnp.int32),
            pltpu.VMEM((C,), jnp.int32),
            pltpu.VMEM((C,), jnp.int32),
            pltpu.VMEM((C, H), jnp.float32),
            pltpu.VMEM((C, H), jnp.float32),
            pltpu.VMEM_SHARED((N, H), jnp.float32),
            pltpu.VMEM_SHARED((n8, H), jnp.float32),
            pltpu.SemaphoreType.DMA,
            pltpu.SemaphoreType.DMA,
            pltpu.SemaphoreType.DMA,
        ],
    )


@functools.cache
def _make_k2(N, H, R, BN):
    nb = N // BN

    def body(agg_ref, deg_ref, root_ref, b1_ref, w2_ref, aq_ref, ak_ref,
             xw_ref, qk_ref):
        agg = agg_ref[0] + agg_ref[1]
        deg = jnp.sum(deg_ref[0] + deg_ref[1], axis=-1, keepdims=True)
        x = agg / jnp.maximum(deg, 1.0) + root_ref[...] + b1_ref[...]
        x = jnp.where(x > 0, x, 0.01 * x)
        w2 = w2_ref[...]
        mq = lax.dot_general(w2, aq_ref[0], (((2,), (0,)), ((), ())),
                             preferred_element_type=jnp.float32)
        mk = lax.dot_general(w2, ak_ref[0], (((2,), (0,)), ((), ())),
                             preferred_element_type=jnp.float32)
        m = jnp.concatenate([mq, mk], axis=0)
        qk_ref[...] = lax.dot_general(x, m, (((1,), (1,)), ((), ())),
                                      preferred_element_type=jnp.float32)
        for r in range(R):
            xw_ref[r] = jnp.dot(x, w2[r], preferred_element_type=jnp.float32)

    return pl.pallas_call(
        body,
        grid=(nb,),
        in_specs=[
            pl.BlockSpec((NC, BN, H), lambda i: (0, i, 0)),
            pl.BlockSpec((NC, BN, L), lambda i: (0, i, 0)),
            pl.BlockSpec((BN, H), lambda i: (i, 0)),
            pl.BlockSpec((1, H), lambda i: (0, 0)),
            pl.BlockSpec((R, H, H), lambda i: (0, 0, 0)),
            pl.BlockSpec((1, H), lambda i: (0, 0)),
            pl.BlockSpec((1, H), lambda i: (0, 0)),
        ],
        out_specs=[
            pl.BlockSpec((R, BN, H), lambda i: (0, i, 0)),
            pl.BlockSpec((BN, 2 * R), lambda i: (i, 0)),
        ],
        out_shape=[
            jax.ShapeDtypeStruct((R, N, H), jnp.float32),
            jax.ShapeDtypeStruct((N, 2 * R), jnp.float32),
        ],
    )


@functools.cache
def _make_k3(N, H, E, R):
    n_chunks = E // C
    base, extra, get_span = _wid_and_span(n_chunks)
    n8 = _pack8_rows(N)

    def body(src_h, rel_h, dst_h, xw_h, qk_h, num_out, den_out,
             sidx_v, ridx_v, didx_v, gidx_v, qidx_v, kidx_v, didx8_v,
             didxs_v, qe_t, ke_t, rows_v, exrow_v, ex_v,
             acc_s, den8_s, sem1, sem2, sem3, sem_idx):
        cid = lax.axis_index("c")
        sid = lax.axis_index("s")
        wid = sid * NC + cid
        start, nj = get_span(wid)
        iota = lax.iota(jnp.int32, L)
        onehot_f = jnp.where(iota == 0, jnp.float32(1.0), jnp.float32(0.0))

        _zero_fill(rows_v, C, H)
        _zero_fill(exrow_v, C, H)

        def zero_slab(r, n):
            pltpu.async_copy(rows_v.at[pl.ds(0, n)], acc_s.at[pl.ds(r, n)],
                             sem1).wait()

        _foreach_slab(sid, N, zero_slab)
        pltpu.async_copy(exrow_v.at[pl.ds(0, n8 // NS)],
                         den8_s.at[pl.ds(sid * (n8 // NS), n8 // NS)],
                         sem1).wait()

        plsc.subcore_barrier()

        def fire_idx(j):
            e0 = (start + j) * C
            pltpu.async_copy(src_h.at[pl.ds(e0, C)], sidx_v, sem_idx)
            pltpu.async_copy(rel_h.at[pl.ds(e0, C)], ridx_v, sem_idx)
            pltpu.async_copy(dst_h.at[pl.ds(e0, C)], didx_v, sem_idx)

        def drain_idx():
            pltpu.make_async_copy(src_h.at[pl.ds(0, C)], sidx_v, sem_idx).wait()
            pltpu.make_async_copy(rel_h.at[pl.ds(0, C)], ridx_v, sem_idx).wait()
            pltpu.make_async_copy(dst_h.at[pl.ds(0, C)], didx_v, sem_idx).wait()

        fire_idx(0)

        def chunk(j, _):
            drain_idx()
            for t in range(C // L):
                sl = pl.ds(t * L, L)
                r = ridx_v[sl]
                d = didx_v[sl]
                gidx_v[sl] = r * N + sidx_v[sl]
                qidx_v[sl] = sidx_v[sl] * (2 * R) + r
                kidx_v[sl] = d * (2 * R) + (r + R)
                didx8_v[sl] = lax.shift_right_logical(d, 3)
                didxs_v[sl] = d
            fire_idx(jnp.minimum(j + 1, nj - 1))
            cp1 = pltpu.async_copy(qk_h.at[qidx_v], qe_t, sem1)
            cp2 = pltpu.async_copy(qk_h.at[kidx_v], ke_t, sem2)
            cp3 = pltpu.async_copy(xw_h.at[gidx_v], rows_v, sem3)
            cp1.wait()
            cp2.wait()
            for t in range(C // L):
                sl = pl.ds(t * L, L)
                lg = qe_t[sl] + ke_t[sl]
                lg = jnp.maximum(lg, 0.2 * lg)
                ex_v[sl] = jnp.exp(lg)
            cp3.wait()

            def scale(g, _):
                exv = ex_v[pl.ds(g * L, L)]
                dm8v = lax.bitwise_and(didxs_v[pl.ds(g * L, L)], 7)
                for p in range(L):
                    sca = exv[p]
                    dm8 = dm8v[p]
                    e = g * L + p
                    for t in range(H // L):
                        sl2 = pl.ds(t * L, L)
                        rows_v[e, sl2] = rows_v[e, sl2] * sca
                        sel = jnp.where(dm8 == t, sca, jnp.float32(0.0))
                        exrow_v[e, sl2] = onehot_f * sel
                return 0

            lax.fori_loop(0, C // L, scale, 0)
            ca = pltpu.async_copy(rows_v, acc_s.at[didxs_v], sem3, add=True)
            cb = pltpu.async_copy(exrow_v, den8_s.at[didx8_v], sem2,
                                  add=True)
            ca.wait()
            cb.wait()
            return 0

        lax.fori_loop(0, nj, chunk, 0)
        drain_idx()

        plsc.subcore_barrier()

        def out_slab(r, n):
            pltpu.async_copy(acc_s.at[pl.ds(r, n)], rows_v.at[pl.ds(0, n)],
                             sem1).wait()
            pltpu.sync_copy(rows_v.at[pl.ds(0, n)],
                            num_out.at[pl.ds(cid * N + r, n)])

        _foreach_slab(sid, N, out_slab)
        r8 = sid * (n8 // NS)
        pltpu.async_copy(den8_s.at[pl.ds(r8, n8 // NS)],
                         exrow_v.at[pl.ds(0, n8 // NS)], sem1).wait()
        pltpu.sync_copy(exrow_v.at[pl.ds(0, n8 // NS)],
                        den_out.at[pl.ds(cid * n8 + r8, n8 // NS)])

    return pl.kernel(
        body,
        out_type=[
            jax.ShapeDtypeStruct((NC * N, H), jnp.float32),
            jax.ShapeDtypeStruct((NC * n8, H), jnp.float32),
        ],
        mesh=_mesh(),
        scratch_types=[
            pltpu.VMEM((C,), jnp.int32),
            pltpu.VMEM((C,), jnp.int32),
            pltpu.VMEM((C,), jnp.int32),
            pltpu.VMEM((C,), jnp.int32),
            pltpu.VMEM((C,), jnp.int32),
            pltpu.VMEM((C,), jnp.int32),
            pltpu.VMEM((C,), jnp.int32),
            pltpu.VMEM((C,), jnp.int32),
            pltpu.VMEM((C,), jnp.float32),
            pltpu.VMEM((C,), jnp.float32),
            pltpu.VMEM((C, H), jnp.float32),
            pltpu.VMEM((C, H), jnp.float32),
            pltpu.VMEM((C,), jnp.float32),
            pltpu.VMEM_SHARED((N, H), jnp.float32),
            pltpu.VMEM_SHARED((n8, H), jnp.float32),
            pltpu.SemaphoreType.DMA,
            pltpu.SemaphoreType.DMA,
            pltpu.SemaphoreType.DMA,
            pltpu.SemaphoreType.DMA,
        ],
    )


@functools.cache
def _make_k4(N, H, BN):
    nb = N // BN

    def body(num_ref, den_ref, b2_ref, out_ref):
        num = num_ref[0] + num_ref[1]
        den = jnp.sum(den_ref[0] + den_ref[1], axis=-1, keepdims=True)
        out_ref[...] = num / jnp.maximum(den, 1e-16) + b2_ref[...]

    return pl.pallas_call(
        body,
        grid=(nb,),
        in_specs=[
            pl.BlockSpec((NC, BN, H), lambda i: (0, i, 0)),
            pl.BlockSpec((NC, BN, L), lambda i: (0, i, 0)),
            pl.BlockSpec((1, H), lambda i: (0, 0)),
        ],
        out_specs=pl.BlockSpec((BN, H), lambda i: (i, 0)),
        out_shape=jax.ShapeDtypeStruct((N, H), jnp.float32),
    )


def kernel(edge_index, edge_type, W1, root1, bias1, W2, att_q, att_k, bias2):
    R, Nw1, H = W1.shape
    N = root1.shape[0]
    E = edge_type.shape[0]
    assert Nw1 == N and E % C == 0 and N % NS == 0 and H % L == 0 and 2 * R == L

    src = edge_index[0]
    dst = edge_index[1]
    w1_flat = W1.reshape(R * N, H)

    n8 = _pack8_rows(N)
    k1 = _make_k1(N, H, E, R)
    aggP, degP = k1(src, edge_type, dst, w1_flat)

    BN = 1000
    k2 = _make_k2(N, H, R, BN)
    xw, qk = k2(aggP.reshape(NC, N, H), degP.reshape(NC, n8 * 8, L),
                root1, bias1.reshape(1, H), W2,
                att_q.reshape(1, H), att_k.reshape(1, H))

    k3 = _make_k3(N, H, E, R)
    numP, denP = k3(src, edge_type, dst, xw.reshape(R * N, H),
                    qk.reshape(N * 2 * R))

    k4 = _make_k4(N, H, BN)
    out = k4(numP.reshape(NC, N, H), denP.reshape(NC, n8 * 8, L),
             bias2.reshape(1, H))
    return out


# speedup vs baseline: 43.3392x; 1.4655x over previous
import functools

import jax
import jax.numpy as jnp
from jax import lax
from jax.experimental import pallas as pl
from jax.experimental.pallas import tpu as pltpu
from jax.experimental.pallas import tpu_sc as plsc

NC = 2
NS = 16
NW = NC * NS
L = 16
C = 128

_mesh = functools.partial(
    plsc.VectorSubcoreMesh,
    core_axis_name="c", subcore_axis_name="s",
    num_cores=NC, num_subcores=NS,
)


def _wid_and_span(n_chunks):
    base, extra = divmod(n_chunks, NW)

    def get(wid):
        start = wid * base + jnp.minimum(wid, extra)
        nj = base + (wid < extra).astype(jnp.int32)
        return start, nj

    return base, extra, get


def _zero_fill(ref, nrows, ncols):
    z = jnp.zeros((L,), jnp.float32)

    def body(r, _):
        for t in range(ncols // L):
            ref[r, pl.ds(t * L, L)] = z
        return 0

    lax.fori_loop(0, nrows, body, 0)


def _row_partition(N):
    rpt = ((N + NS - 1) // NS + 7) // 8 * 8
    last = N - (NS - 1) * rpt
    assert 0 < last <= rpt and last % 8 == 0
    full = last // C
    return rpt, full, rpt - full * C, last - full * C


def _foreach_slab(sid, N, cb):
    rpt, full, tail_main, tail_last = _row_partition(N)
    r0 = sid * rpt
    for i in range(full):
        cb(r0 + i * C, C)
    if tail_main:
        @pl.when(sid < NS - 1)
        def _():
            cb(r0 + full * C, tail_main)
    if tail_last:
        @pl.when(sid == NS - 1)
        def _():
            cb(r0 + full * C, tail_last)


def _pack8_rows(N):
    per = NS * 8
    return ((N + 8 - 1) // 8 + per - 1) // per * per


@functools.cache
def _make_k1(N, H, E, R):
    n_chunks = E // C
    base, extra, get_span = _wid_and_span(n_chunks)
    n8 = _pack8_rows(N)

    def body(src_h, rel_h, dst_h, w1_h, agg_out, deg_out,
             sidx_v, ridx_v, didx_v, gidx_v, didxs_v,
             rows_v, ones_t, acc_s, deg_s, sem, sem2, sem_idx):
        cid = lax.axis_index("c")
        sid = lax.axis_index("s")
        wid = sid * NC + cid
        start, nj = get_span(wid)
        _zero_fill(rows_v, C, H)
        zf = jnp.zeros((L,), jnp.float32)
        for t in range(C // L):
            ones_t[pl.ds(t * L, L)] = zf

        def zero_slab(r, n):
            pltpu.async_copy(rows_v.at[pl.ds(0, n)], acc_s.at[pl.ds(r, n)],
                             sem).wait()
            pltpu.async_copy(ones_t.at[pl.ds(0, n)], deg_s.at[pl.ds(r, n)],
                             sem).wait()

        _foreach_slab(sid, N, zero_slab)
        onesv = zf + 1.0
        for t in range(C // L):
            ones_t[pl.ds(t * L, L)] = onesv

        plsc.subcore_barrier()

        def fire_idx(j):
            e0 = (start + j) * C
            pltpu.async_copy(src_h.at[pl.ds(e0, C)], sidx_v, sem_idx)
            pltpu.async_copy(rel_h.at[pl.ds(e0, C)], ridx_v, sem_idx)
            pltpu.async_copy(dst_h.at[pl.ds(e0, C)], didx_v, sem_idx)

        def drain_idx():
            pltpu.make_async_copy(src_h.at[pl.ds(0, C)], sidx_v, sem_idx).wait()
            pltpu.make_async_copy(rel_h.at[pl.ds(0, C)], ridx_v, sem_idx).wait()
            pltpu.make_async_copy(dst_h.at[pl.ds(0, C)], didx_v, sem_idx).wait()

        fire_idx(0)

        def chunk(j, _):
            drain_idx()
            for t in range(C // L):
                sl = pl.ds(t * L, L)
                gidx_v[sl] = ridx_v[sl] * N + sidx_v[sl]
                didxs_v[sl] = didx_v[sl]
            fire_idx(jnp.minimum(j + 1, nj - 1))
            cpw = pltpu.async_copy(w1_h.at[gidx_v], rows_v, sem)
            cpw.wait()
            ca = pltpu.async_copy(rows_v, acc_s.at[didxs_v], sem, add=True)
            cb = pltpu.async_copy(ones_t, deg_s.at[didxs_v], sem2, add=True)
            ca.wait()
            cb.wait()
            return 0

        lax.fori_loop(0, nj, chunk, 0)
        drain_idx()

        plsc.subcore_barrier()

        def out_slab(r, n):
            pltpu.async_copy(acc_s.at[pl.ds(r, n)], rows_v.at[pl.ds(0, n)],
                             sem).wait()
            pltpu.sync_copy(rows_v.at[pl.ds(0, n)],
                            agg_out.at[pl.ds(cid * N + r, n)])

        def deg_slab(r, n):
            pltpu.async_copy(deg_s.at[pl.ds(r, n)], ones_t.at[pl.ds(0, n)],
                             sem).wait()
            pltpu.sync_copy(ones_t.at[pl.ds(0, n)],
                            deg_out.at[pl.ds(cid * N + r, n)])

        _foreach_slab(sid, N, out_slab)
        _foreach_slab(sid, N, deg_slab)

    return pl.kernel(
        body,
        out_type=[
            jax.ShapeDtypeStruct((NC * N, H), jnp.float32),
            jax.ShapeDtypeStruct((NC * N,), jnp.float32),
        ],
        mesh=_mesh(),
        scratch_types=[
            pltpu.VMEM((C,), jnp.int32),
            pltpu.VMEM((C,), jnp.int32),
            pltpu.VMEM((C,), jnp.int32),
            pltpu.VMEM((C,), jnp.int32),
            pltpu.VMEM((C,), jnp.int32),
            pltpu.VMEM((C, H), jnp.float32),
            pltpu.VMEM((C,), jnp.float32),
            pltpu.VMEM_SHARED((N, H), jnp.float32),
            pltpu.VMEM_SHARED((N,), jnp.float32),
            pltpu.SemaphoreType.DMA,
            pltpu.SemaphoreType.DMA,
            pltpu.SemaphoreType.DMA,
        ],
    )


@functools.cache
def _make_k2(N, H, R, BN):
    nb = N // BN

    def body(agg_ref, deg_ref, root_ref, b1_ref, w2_ref, aq_ref, ak_ref,
             xw_ref, qk_ref):
        agg = agg_ref[0] + agg_ref[1]
        deg = deg_ref[0] + deg_ref[1]
        x = agg / jnp.maximum(deg, 1.0) + root_ref[...] + b1_ref[...]
        x = jnp.where(x > 0, x, 0.01 * x)
        w2 = w2_ref[...]
        mq = lax.dot_general(w2, aq_ref[0], (((2,), (0,)), ((), ())),
                             preferred_element_type=jnp.float32)
        mk = lax.dot_general(w2, ak_ref[0], (((2,), (0,)), ((), ())),
                             preferred_element_type=jnp.float32)
        m = jnp.concatenate([mq, mk], axis=0)
        qk_ref[...] = lax.dot_general(x, m, (((1,), (1,)), ((), ())),
                                      preferred_element_type=jnp.float32)
        for r in range(R):
            xw_ref[r] = jnp.dot(x, w2[r], preferred_element_type=jnp.float32)

    return pl.pallas_call(
        body,
        grid=(nb,),
        in_specs=[
            pl.BlockSpec((NC, BN, H), lambda i: (0, i, 0)),
            pl.BlockSpec((NC, BN, 1), lambda i: (0, i, 0)),
            pl.BlockSpec((BN, H), lambda i: (i, 0)),
            pl.BlockSpec((1, H), lambda i: (0, 0)),
            pl.BlockSpec((R, H, H), lambda i: (0, 0, 0)),
            pl.BlockSpec((1, H), lambda i: (0, 0)),
            pl.BlockSpec((1, H), lambda i: (0, 0)),
        ],
        out_specs=[
            pl.BlockSpec((R, BN, H), lambda i: (0, i, 0)),
            pl.BlockSpec((BN, 2 * R), lambda i: (i, 0)),
        ],
        out_shape=[
            jax.ShapeDtypeStruct((R, N, H), jnp.float32),
            jax.ShapeDtypeStruct((N, 2 * R), jnp.float32),
        ],
    )


@functools.cache
def _make_k3(N, H, E, R):
    n_chunks = E // C
    base, extra, get_span = _wid_and_span(n_chunks)
    n8 = _pack8_rows(N)

    def body(src_h, rel_h, dst_h, xw_h, qk_h, num_out, den_out,
             sidx_v, ridx_v, didx_v, gidx_v, qidx_v, kidx_v,
             didxs_v, qe_t, ke_t, rows_v, ex_v,
             acc_s, den_s, sem1, sem2, sem3, sem_idx):
        cid = lax.axis_index("c")
        sid = lax.axis_index("s")
        wid = sid * NC + cid
        start, nj = get_span(wid)
        _zero_fill(rows_v, C, H)
        zf = jnp.zeros((L,), jnp.float32)
        for t in range(C // L):
            ex_v[pl.ds(t * L, L)] = zf

        def zero_slab(r, n):
            pltpu.async_copy(rows_v.at[pl.ds(0, n)], acc_s.at[pl.ds(r, n)],
                             sem1).wait()
            pltpu.async_copy(ex_v.at[pl.ds(0, n)], den_s.at[pl.ds(r, n)],
                             sem1).wait()

        _foreach_slab(sid, N, zero_slab)

        plsc.subcore_barrier()

        def fire_idx(j):
            e0 = (start + j) * C
            pltpu.async_copy(src_h.at[pl.ds(e0, C)], sidx_v, sem_idx)
            pltpu.async_copy(rel_h.at[pl.ds(e0, C)], ridx_v, sem_idx)
            pltpu.async_copy(dst_h.at[pl.ds(e0, C)], didx_v, sem_idx)

        def drain_idx():
            pltpu.make_async_copy(src_h.at[pl.ds(0, C)], sidx_v, sem_idx).wait()
            pltpu.make_async_copy(rel_h.at[pl.ds(0, C)], ridx_v, sem_idx).wait()
            pltpu.make_async_copy(dst_h.at[pl.ds(0, C)], didx_v, sem_idx).wait()

        fire_idx(0)

        def chunk(j, _):
            drain_idx()
            for t in range(C // L):
                sl = pl.ds(t * L, L)
                r = ridx_v[sl]
                d = didx_v[sl]
                gidx_v[sl] = r * N + sidx_v[sl]
                qidx_v[sl] = sidx_v[sl] * (2 * R) + r
                kidx_v[sl] = d * (2 * R) + (r + R)
                didxs_v[sl] = d
            fire_idx(jnp.minimum(j + 1, nj - 1))
            cp1 = pltpu.async_copy(qk_h.at[qidx_v], qe_t, sem1)
            cp2 = pltpu.async_copy(qk_h.at[kidx_v], ke_t, sem2)
            cp3 = pltpu.async_copy(xw_h.at[gidx_v], rows_v, sem3)
            cp1.wait()
            cp2.wait()
            for t in range(C // L):
                sl = pl.ds(t * L, L)
                lg = qe_t[sl] + ke_t[sl]
                lg = jnp.maximum(lg, 0.2 * lg)
                ex_v[sl] = jnp.exp(lg)
            cp3.wait()

            def scale(g, _):
                exv = ex_v[pl.ds(g * L, L)]
                for p in range(L):
                    sca = exv[p]
                    e = g * L + p
                    for t in range(H // L):
                        sl2 = pl.ds(t * L, L)
                        rows_v[e, sl2] = rows_v[e, sl2] * sca
                return 0

            lax.fori_loop(0, C // L, scale, 0)
            ca = pltpu.async_copy(rows_v, acc_s.at[didxs_v], sem3, add=True)
            cb = pltpu.async_copy(ex_v, den_s.at[didxs_v], sem2, add=True)
            ca.wait()
            cb.wait()
            return 0

        lax.fori_loop(0, nj, chunk, 0)
        drain_idx()

        plsc.subcore_barrier()

        def out_slab(r, n):
            pltpu.async_copy(acc_s.at[pl.ds(r, n)], rows_v.at[pl.ds(0, n)],
                             sem1).wait()
            pltpu.sync_copy(rows_v.at[pl.ds(0, n)],
                            num_out.at[pl.ds(cid * N + r, n)])

        def den_slab(r, n):
            pltpu.async_copy(den_s.at[pl.ds(r, n)], ex_v.at[pl.ds(0, n)],
                             sem1).wait()
            pltpu.sync_copy(ex_v.at[pl.ds(0, n)],
                            den_out.at[pl.ds(cid * N + r, n)])

        _foreach_slab(sid, N, out_slab)
        _foreach_slab(sid, N, den_slab)

    return pl.kernel(
        body,
        out_type=[
            jax.ShapeDtypeStruct((NC * N, H), jnp.float32),
            jax.ShapeDtypeStruct((NC * N,), jnp.float32),
        ],
        mesh=_mesh(),
        scratch_types=[
            pltpu.VMEM((C,), jnp.int32),
            pltpu.VMEM((C,), jnp.int32),
            pltpu.VMEM((C,), jnp.int32),
            pltpu.VMEM((C,), jnp.int32),
            pltpu.VMEM((C,), jnp.int32),
            pltpu.VMEM((C,), jnp.int32),
            pltpu.VMEM((C,), jnp.int32),
            pltpu.VMEM((C,), jnp.float32),
            pltpu.VMEM((C,), jnp.float32),
            pltpu.VMEM((C, H), jnp.float32),
            pltpu.VMEM((C,), jnp.float32),
            pltpu.VMEM_SHARED((N, H), jnp.float32),
            pltpu.VMEM_SHARED((N,), jnp.float32),
            pltpu.SemaphoreType.DMA,
            pltpu.SemaphoreType.DMA,
            pltpu.SemaphoreType.DMA,
            pltpu.SemaphoreType.DMA,
        ],
    )


@functools.cache
def _make_k4(N, H, BN):
    nb = N // BN

    def body(num_ref, den_ref, b2_ref, out_ref):
        num = num_ref[0] + num_ref[1]
        den = den_ref[0] + den_ref[1]
        out_ref[...] = num / jnp.maximum(den, 1e-16) + b2_ref[...]

    return pl.pallas_call(
        body,
        grid=(nb,),
        in_specs=[
            pl.BlockSpec((NC, BN, H), lambda i: (0, i, 0)),
            pl.BlockSpec((NC, BN, 1), lambda i: (0, i, 0)),
            pl.BlockSpec((1, H), lambda i: (0, 0)),
        ],
        out_specs=pl.BlockSpec((BN, H), lambda i: (i, 0)),
        out_shape=jax.ShapeDtypeStruct((N, H), jnp.float32),
    )


def kernel(edge_index, edge_type, W1, root1, bias1, W2, att_q, att_k, bias2):
    R, Nw1, H = W1.shape
    N = root1.shape[0]
    E = edge_type.shape[0]
    assert Nw1 == N and E % C == 0 and N % NS == 0 and H % L == 0 and 2 * R == L

    src = edge_index[0]
    dst = edge_index[1]
    w1_flat = W1.reshape(R * N, H)

    k1 = _make_k1(N, H, E, R)
    aggP, degP = k1(src, edge_type, dst, w1_flat)

    BN = 1000
    k2 = _make_k2(N, H, R, BN)
    xw, qk = k2(aggP.reshape(NC, N, H), degP.reshape(NC, N, 1),
                root1, bias1.reshape(1, H), W2,
                att_q.reshape(1, H), att_k.reshape(1, H))

    k3 = _make_k3(N, H, E, R)
    numP, denP = k3(src, edge_type, dst, xw.reshape(R * N, H),
                    qk.reshape(N * 2 * R))

    k4 = _make_k4(N, H, BN)
    out = k4(numP.reshape(NC, N, H), denP.reshape(NC, N, 1),
             bias2.reshape(1, H))
    return out


# speedup vs baseline: 43.3942x; 1.0013x over previous
import functools

import jax
import jax.numpy as jnp
from jax import lax
from jax.experimental import pallas as pl
from jax.experimental.pallas import tpu as pltpu
from jax.experimental.pallas import tpu_sc as plsc

NC = 2
NS = 16
NW = NC * NS
L = 16
C = 128

_mesh = functools.partial(
    plsc.VectorSubcoreMesh,
    core_axis_name="c", subcore_axis_name="s",
    num_cores=NC, num_subcores=NS,
)


def _wid_and_span(n_chunks):
    base, extra = divmod(n_chunks, NW)

    def get(wid):
        start = wid * base + jnp.minimum(wid, extra)
        nj = base + (wid < extra).astype(jnp.int32)
        return start, nj

    return base, extra, get


def _zero_fill(ref, nrows, ncols):
    z = jnp.zeros((L,), jnp.float32)

    def body(r, _):
        for t in range(ncols // L):
            ref[r, pl.ds(t * L, L)] = z
        return 0

    lax.fori_loop(0, nrows, body, 0)


def _row_partition(N):
    rpt = ((N + NS - 1) // NS + 7) // 8 * 8
    last = N - (NS - 1) * rpt
    assert 0 < last <= rpt and last % 8 == 0
    full = last // C
    return rpt, full, rpt - full * C, last - full * C


def _foreach_slab(sid, N, cb):
    rpt, full, tail_main, tail_last = _row_partition(N)
    r0 = sid * rpt
    for i in range(full):
        cb(r0 + i * C, C)
    if tail_main:
        @pl.when(sid < NS - 1)
        def _():
            cb(r0 + full * C, tail_main)
    if tail_last:
        @pl.when(sid == NS - 1)
        def _():
            cb(r0 + full * C, tail_last)


@functools.cache
def _make_k1(N, H, E, R):
    n_chunks = E // C
    base, extra, get_span = _wid_and_span(n_chunks)

    def body(src_h, rel_h, dst_h, w1_h, agg_out, deg_out,
             sidx_v, ridx_v, didx_v, gidx_v, didxs_v,
             rows_v, ones_t, acc_s, deg_s, sem, sem2, sem_idx):
        cid = lax.axis_index("c")
        sid = lax.axis_index("s")
        wid = sid * NC + cid
        start, nj = get_span(wid)
        _zero_fill(rows_v, C, H)
        zf = jnp.zeros((L,), jnp.float32)
        for t in range(C // L):
            ones_t[pl.ds(t * L, L)] = zf

        def zero_slab(r, n):
            pltpu.async_copy(rows_v.at[pl.ds(0, n)], acc_s.at[pl.ds(r, n)],
                             sem).wait()
            pltpu.async_copy(ones_t.at[pl.ds(0, n)], deg_s.at[pl.ds(r, n)],
                             sem).wait()

        _foreach_slab(sid, N, zero_slab)
        onesv = zf + 1.0
        for t in range(C // L):
            ones_t[pl.ds(t * L, L)] = onesv

        plsc.subcore_barrier()

        def fire_idx(j):
            e0 = (start + j) * C
            pltpu.async_copy(src_h.at[pl.ds(e0, C)], sidx_v, sem_idx)
            pltpu.async_copy(rel_h.at[pl.ds(e0, C)], ridx_v, sem_idx)
            pltpu.async_copy(dst_h.at[pl.ds(e0, C)], didx_v, sem_idx)

        def drain_idx():
            pltpu.make_async_copy(src_h.at[pl.ds(0, C)], sidx_v, sem_idx).wait()
            pltpu.make_async_copy(rel_h.at[pl.ds(0, C)], ridx_v, sem_idx).wait()
            pltpu.make_async_copy(dst_h.at[pl.ds(0, C)], didx_v, sem_idx).wait()

        fire_idx(0)

        def chunk(j, _):
            drain_idx()
            for t in range(C // L):
                sl = pl.ds(t * L, L)
                gidx_v[sl] = ridx_v[sl] * N + sidx_v[sl]
                didxs_v[sl] = didx_v[sl]
            fire_idx(jnp.minimum(j + 1, nj - 1))
            cpw = pltpu.async_copy(w1_h.at[gidx_v], rows_v, sem)
            cpw.wait()
            ca = pltpu.async_copy(rows_v, acc_s.at[didxs_v], sem, add=True)
            cb = pltpu.async_copy(ones_t, deg_s.at[didxs_v], sem2, add=True)
            ca.wait()
            cb.wait()
            return 0

        lax.fori_loop(0, nj, chunk, 0)
        drain_idx()

        plsc.subcore_barrier()

        def out_slab(r, n):
            pltpu.async_copy(acc_s.at[pl.ds(r, n)], rows_v.at[pl.ds(0, n)],
                             sem).wait()
            pltpu.sync_copy(rows_v.at[pl.ds(0, n)],
                            agg_out.at[pl.ds(cid * N + r, n)])

        def deg_slab(r, n):
            pltpu.async_copy(deg_s.at[pl.ds(r, n)], ones_t.at[pl.ds(0, n)],
                             sem).wait()
            pltpu.sync_copy(ones_t.at[pl.ds(0, n)],
                            deg_out.at[pl.ds(cid * N + r, n)])

        _foreach_slab(sid, N, out_slab)
        _foreach_slab(sid, N, deg_slab)

    return pl.kernel(
        body,
        out_type=[
            jax.ShapeDtypeStruct((NC * N, H), jnp.float32),
            jax.ShapeDtypeStruct((NC * N,), jnp.float32),
        ],
        mesh=_mesh(),
        scratch_types=[
            pltpu.VMEM((C,), jnp.int32),
            pltpu.VMEM((C,), jnp.int32),
            pltpu.VMEM((C,), jnp.int32),
            pltpu.VMEM((C,), jnp.int32),
            pltpu.VMEM((C,), jnp.int32),
            pltpu.VMEM((C, H), jnp.float32),
            pltpu.VMEM((C,), jnp.float32),
            pltpu.VMEM_SHARED((N, H), jnp.float32),
            pltpu.VMEM_SHARED((N,), jnp.float32),
            pltpu.SemaphoreType.DMA,
            pltpu.SemaphoreType.DMA,
            pltpu.SemaphoreType.DMA,
        ],
    )


@functools.cache
def _make_k2(N, H, R, BN):
    nb = N // BN

    def body(agg_ref, deg_ref, root_ref, b1_ref, w2_ref, aq_ref, ak_ref,
             xw_ref, qk_ref):
        agg = agg_ref[0] + agg_ref[1]
        deg = deg_ref[0] + deg_ref[1]
        x = agg / jnp.maximum(deg, 1.0) + root_ref[...] + b1_ref[...]
        x = jnp.where(x > 0, x, 0.01 * x)
        w2 = w2_ref[...]
        mq = lax.dot_general(w2, aq_ref[0], (((2,), (0,)), ((), ())),
                             preferred_element_type=jnp.float32)
        mk = lax.dot_general(w2, ak_ref[0], (((2,), (0,)), ((), ())),
                             preferred_element_type=jnp.float32)
        m = jnp.concatenate([mq, mk], axis=0)
        qk_ref[...] = lax.dot_general(x, m, (((1,), (1,)), ((), ())),
                                      preferred_element_type=jnp.float32)
        for r in range(R):
            xw_ref[r] = jnp.dot(x, w2[r], preferred_element_type=jnp.float32)

    return pl.pallas_call(
        body,
        grid=(nb,),
        in_specs=[
            pl.BlockSpec((NC, BN, H), lambda i: (0, i, 0)),
            pl.BlockSpec((NC, BN, 1), lambda i: (0, i, 0)),
            pl.BlockSpec((BN, H), lambda i: (i, 0)),
            pl.BlockSpec((1, H), lambda i: (0, 0)),
            pl.BlockSpec((R, H, H), lambda i: (0, 0, 0)),
            pl.BlockSpec((1, H), lambda i: (0, 0)),
            pl.BlockSpec((1, H), lambda i: (0, 0)),
        ],
        out_specs=[
            pl.BlockSpec((R, BN, H), lambda i: (0, i, 0)),
            pl.BlockSpec((BN, 2 * R), lambda i: (i, 0)),
        ],
        out_shape=[
            jax.ShapeDtypeStruct((R, N, H), jnp.float32),
            jax.ShapeDtypeStruct((N, 2 * R), jnp.float32),
        ],
    )


@functools.cache
def _make_k3(N, H, E, R):
    n_chunks = E // C
    base, extra, get_span = _wid_and_span(n_chunks)

    def body(src_h, rel_h, dst_h, xw_h, qk_h, num_out, den_out,
             sidx_v, ridx_v, didx_v, gidx_v, qidx_v, kidx_v,
             didxs_v, qe_t, ke_t, rows_v, ex_v,
             acc_s, den_s, sem1, sem2, sem3, sem_idx):
        cid = lax.axis_index("c")
        sid = lax.axis_index("s")
        wid = sid * NC + cid
        start, nj = get_span(wid)
        _zero_fill(rows_v, C, H)
        zf = jnp.zeros((L,), jnp.float32)
        for t in range(C // L):
            ex_v[pl.ds(t * L, L)] = zf

        def zero_slab(r, n):
            pltpu.async_copy(rows_v.at[pl.ds(0, n)], acc_s.at[pl.ds(r, n)],
                             sem1).wait()
            pltpu.async_copy(ex_v.at[pl.ds(0, n)], den_s.at[pl.ds(r, n)],
                             sem1).wait()

        _foreach_slab(sid, N, zero_slab)

        plsc.subcore_barrier()

        def fire_idx(j):
            e0 = (start + j) * C
            pltpu.async_copy(src_h.at[pl.ds(e0, C)], sidx_v, sem_idx)
            pltpu.async_copy(rel_h.at[pl.ds(e0, C)], ridx_v, sem_idx)
            pltpu.async_copy(dst_h.at[pl.ds(e0, C)], didx_v, sem_idx)

        def drain_idx():
            pltpu.make_async_copy(src_h.at[pl.ds(0, C)], sidx_v, sem_idx).wait()
            pltpu.make_async_copy(rel_h.at[pl.ds(0, C)], ridx_v, sem_idx).wait()
            pltpu.make_async_copy(dst_h.at[pl.ds(0, C)], didx_v, sem_idx).wait()

        fire_idx(0)

        def chunk(j, _):
            drain_idx()
            for t in range(C // L):
                sl = pl.ds(t * L, L)
                r = ridx_v[sl]
                d = didx_v[sl]
                gidx_v[sl] = r * N + sidx_v[sl]
                qidx_v[sl] = sidx_v[sl] * (2 * R) + r
                kidx_v[sl] = d * (2 * R) + (r + R)
                didxs_v[sl] = d
            fire_idx(jnp.minimum(j + 1, nj - 1))
            cp1 = pltpu.async_copy(qk_h.at[qidx_v], qe_t, sem1)
            cp2 = pltpu.async_copy(qk_h.at[kidx_v], ke_t, sem2)
            cp3 = pltpu.async_copy(xw_h.at[gidx_v], rows_v, sem3)
            cp1.wait()
            cp2.wait()
            for t in range(C // L):
                sl = pl.ds(t * L, L)
                lg = qe_t[sl] + ke_t[sl]
                lg = jnp.maximum(lg, 0.2 * lg)
                ex_v[sl] = jnp.exp(lg)
            cp3.wait()

            def scale(g, _):
                exv = ex_v[pl.ds(g * L, L)]
                for p in range(L):
                    sca = exv[p]
                    e = g * L + p
                    for t in range(H // L):
                        sl2 = pl.ds(t * L, L)
                        rows_v[e, sl2] = rows_v[e, sl2] * sca
                return 0

            lax.fori_loop(0, C // L, scale, 0)
            ca = pltpu.async_copy(rows_v, acc_s.at[didxs_v], sem3, add=True)
            cb = pltpu.async_copy(ex_v, den_s.at[didxs_v], sem2, add=True)
            ca.wait()
            cb.wait()
            return 0

        lax.fori_loop(0, nj, chunk, 0)
        drain_idx()

        plsc.subcore_barrier()

        def out_slab(r, n):
            pltpu.async_copy(acc_s.at[pl.ds(r, n)], rows_v.at[pl.ds(0, n)],
                             sem1).wait()
            pltpu.sync_copy(rows_v.at[pl.ds(0, n)],
                            num_out.at[pl.ds(cid * N + r, n)])

        def den_slab(r, n):
            pltpu.async_copy(den_s.at[pl.ds(r, n)], ex_v.at[pl.ds(0, n)],
                             sem1).wait()
            pltpu.sync_copy(ex_v.at[pl.ds(0, n)],
                            den_out.at[pl.ds(cid * N + r, n)])

        _foreach_slab(sid, N, out_slab)
        _foreach_slab(sid, N, den_slab)

    return pl.kernel(
        body,
        out_type=[
            jax.ShapeDtypeStruct((NC * N, H), jnp.float32),
            jax.ShapeDtypeStruct((NC * N,), jnp.float32),
        ],
        mesh=_mesh(),
        scratch_types=[
            pltpu.VMEM((C,), jnp.int32),
            pltpu.VMEM((C,), jnp.int32),
            pltpu.VMEM((C,), jnp.int32),
            pltpu.VMEM((C,), jnp.int32),
            pltpu.VMEM((C,), jnp.int32),
            pltpu.VMEM((C,), jnp.int32),
            pltpu.VMEM((C,), jnp.int32),
            pltpu.VMEM((C,), jnp.float32),
            pltpu.VMEM((C,), jnp.float32),
            pltpu.VMEM((C, H), jnp.float32),
            pltpu.VMEM((C,), jnp.float32),
            pltpu.VMEM_SHARED((N, H), jnp.float32),
            pltpu.VMEM_SHARED((N,), jnp.float32),
            pltpu.SemaphoreType.DMA,
            pltpu.SemaphoreType.DMA,
            pltpu.SemaphoreType.DMA,
            pltpu.SemaphoreType.DMA,
        ],
    )


@functools.cache
def _make_k4(N, H, BN):
    nb = N // BN

    def body(num_ref, den_ref, b2_ref, out_ref):
        num = num_ref[0] + num_ref[1]
        den = den_ref[0] + den_ref[1]
        out_ref[...] = num / jnp.maximum(den, 1e-16) + b2_ref[...]

    return pl.pallas_call(
        body,
        grid=(nb,),
        in_specs=[
            pl.BlockSpec((NC, BN, H), lambda i: (0, i, 0)),
            pl.BlockSpec((NC, BN, 1), lambda i: (0, i, 0)),
            pl.BlockSpec((1, H), lambda i: (0, 0)),
        ],
        out_specs=pl.BlockSpec((BN, H), lambda i: (i, 0)),
        out_shape=jax.ShapeDtypeStruct((N, H), jnp.float32),
    )


def kernel(edge_index, edge_type, W1, root1, bias1, W2, att_q, att_k, bias2):
    R, Nw1, H = W1.shape
    N = root1.shape[0]
    E = edge_type.shape[0]
    assert Nw1 == N and E % C == 0 and N % NS == 0 and H % L == 0 and 2 * R == L

    src = edge_index[0]
    dst = edge_index[1]
    w1_flat = W1.reshape(R * N, H)

    k1 = _make_k1(N, H, E, R)
    aggP, degP = k1(src, edge_type, dst, w1_flat)

    BN = 1000
    k2 = _make_k2(N, H, R, BN)
    xw, qk = k2(aggP.reshape(NC, N, H), degP.reshape(NC, N, 1),
                root1, bias1.reshape(1, H), W2,
                att_q.reshape(1, H), att_k.reshape(1, H))

    k3 = _make_k3(N, H, E, R)
    numP, denP = k3(src, edge_type, dst, xw.reshape(R * N, H),
                    qk.reshape(N * 2 * R))

    k4 = _make_k4(N, H, BN)
    out = k4(numP.reshape(NC, N, H), denP.reshape(NC, N, 1),
             bias2.reshape(1, H))
    return out
